# baseline (speedup 1.0000x reference)
import functools

import jax
import jax.numpy as jnp
from jax import lax
from jax.experimental import pallas as pl
from jax.experimental.pallas import tpu as pltpu
from jax.experimental.pallas import tpu_sc as plsc

N = 10000
E = 320000
D = 128
T64 = 64
BN_EPS = 1e-5
NC, NS, L = 2, 16, 16
NW = NC * NS
CKP = 128
NCKP = 80
E_PAD = NW * NCKP * CKP
ZCK = 80
NODE_CKS = N // ZCK
NP = N + 128

_mesh = plsc.VectorSubcoreMesh(core_axis_name="c", subcore_axis_name="s")


@functools.partial(
    pl.kernel,
    out_type=(
        jax.ShapeDtypeStruct((N,), jnp.float32),
        jax.ShapeDtypeStruct((N,), jnp.float32),
        jax.ShapeDtypeStruct((N, D), jnp.float32),
    ),
    mesh=_mesh,
    scratch_types=[
        pltpu.VMEM((NCKP, CKP), jnp.int32),
        pltpu.VMEM((CKP,), jnp.float32),
        pltpu.VMEM((ZCK,), jnp.int32),
        pltpu.VMEM((ZCK, D), jnp.float32),
        pltpu.VMEM((N,), jnp.float32),
        pltpu.VMEM_SHARED((NP,), jnp.float32),
        pltpu.SemaphoreType.DMA,
        pltpu.SemaphoreType.DMA,
    ],
)
def _sc_deg_emb(srcd_hbm, xcol_hbm, emb_hbm, zeros_n_hbm,
                deg0_hbm, deg1_hbm, h0_hbm,
                src_all, ones_v, nid_v, rows_v, bounce_v, sh_deg, sems, semg):
    cid = lax.axis_index("c")
    sid = lax.axis_index("s")
    wid = cid * NS + sid

    pltpu.sync_copy(srcd_hbm.at[wid], src_all)

    @pl.when(sid == 0)
    def _():
        pltpu.sync_copy(zeros_n_hbm, bounce_v)
        pltpu.sync_copy(bounce_v, sh_deg.at[pl.ds(0, N)])

    for k in range(CKP // L):
        ones_v[pl.ds(k * L, L)] = jnp.ones((L,), jnp.float32)
    plsc.subcore_barrier()

    for grp in range(NCKP // 16):
        def fire(i, carry):
            c = grp * 16 + i
            pltpu.async_copy(ones_v, sh_deg.at[src_all.at[c]], sems, add=True)
            return carry

        lax.fori_loop(0, 16, fire, 0)

        def drain(i, carry):
            pltpu.make_async_copy(ones_v, sh_deg.at[src_all.at[0]], sems).wait()
            return carry

        lax.fori_loop(0, 16, drain, 0)

    for j in range((NODE_CKS + NW - 1) // NW):
        c = wid + j * NW

        @pl.when(c < NODE_CKS)
        def _():
            pltpu.sync_copy(xcol_hbm.at[pl.ds(c * ZCK, ZCK)], nid_v)
            pltpu.async_copy(emb_hbm.at[nid_v], rows_v, semg).wait()
            pltpu.sync_copy(rows_v, h0_hbm.at[pl.ds(c * ZCK, ZCK)])

    plsc.subcore_barrier()

    @pl.when((sid == 0) & (cid == 0))
    def _():
        pltpu.sync_copy(sh_deg.at[pl.ds(0, N)], bounce_v)
        pltpu.sync_copy(bounce_v, deg0_hbm)

    @pl.when((sid == 0) & (cid == 1))
    def _():
        pltpu.sync_copy(sh_deg.at[pl.ds(0, N)], bounce_v)
        pltpu.sync_copy(bounce_v, deg1_hbm)


@functools.partial(
    pl.kernel,
    out_type=(
        jax.ShapeDtypeStruct((N * T64,), jnp.float32),
        jax.ShapeDtypeStruct((N * T64,), jnp.float32),
    ),
    mesh=_mesh,
    scratch_types=[
        pltpu.VMEM((4, CKP), jnp.int32),
        pltpu.VMEM((4, CKP), jnp.int32),
        pltpu.VMEM((4, CKP), jnp.float32),
        pltpu.VMEM((N * T64 // NS,), jnp.float32),
        pltpu.VMEM_SHARED((NP * T64,), jnp.float32),
        [pltpu.SemaphoreType.DMA] * 4,
        [pltpu.SemaphoreType.DMA] * 4,
        [pltpu.SemaphoreType.DMA] * 4,
    ],
)
def _sc_ctab(srcg_hbm, cidx_hbm, dinv_hbm, zeros_c_hbm,
             cp0_hbm, cp1_hbm,
             src_ring, dst_ring, val_ring, bounce_v, sh_c,
             semi, semg, sems):
    cid = lax.axis_index("c")
    sid = lax.axis_index("s")
    wid = cid * NS + sid
    zlen = N * T64 // NS

    pltpu.sync_copy(zeros_c_hbm.at[pl.ds(sid * zlen, zlen)], bounce_v)
    pltpu.sync_copy(bounce_v, sh_c.at[pl.ds(sid * zlen, zlen)])
    plsc.subcore_barrier()

    def l_start(c, q):
        pltpu.async_copy(srcg_hbm.at[wid].at[c], src_ring.at[q], semi[q])
        pltpu.async_copy(cidx_hbm.at[wid].at[c], dst_ring.at[q], semi[q])

    def l_wait(c, q):
        for _ in range(2):
            pltpu.make_async_copy(srcg_hbm.at[wid].at[c], src_ring.at[q],
                                  semi[q]).wait()

    def g_start(c, q):
        pltpu.async_copy(dinv_hbm.at[src_ring.at[q]], val_ring.at[q], semg[q])

    def g_wait(c, q):
        pltpu.make_async_copy(dinv_hbm.at[src_ring.at[q]], val_ring.at[q],
                              semg[q]).wait()

    def s_start(c, q):
        pltpu.async_copy(val_ring.at[q], sh_c.at[dst_ring.at[q]], sems[q],
                         add=True)

    def s_wait(c, q):
        pltpu.make_async_copy(val_ring.at[q], sh_c.at[dst_ring.at[q]],
                              sems[q]).wait()

    l_start(0, 0)
    l_start(1, 1)
    l_wait(0, 0)
    g_start(0, 0)

    def body(g, carry):
        for q in range(4):
            c = g * 4 + q

            @pl.when(c - 2 >= 0)
            def _():
                s_wait(c - 2, (q + 2) % 4)

            @pl.when(c + 2 < NCKP)
            def _():
                l_start(c + 2, (q + 2) % 4)

            @pl.when(c + 1 < NCKP)
            def _():
                l_wait(c + 1, (q + 1) % 4)
                g_start(c + 1, (q + 1) % 4)

            g_wait(c, q)
            s_start(c, q)
        return carry

    lax.fori_loop(0, NCKP // 4, body, 0)
    s_wait(NCKP - 2, (NCKP - 2) % 4)
    s_wait(NCKP - 1, (NCKP - 1) % 4)

    plsc.subcore_barrier()
    pltpu.sync_copy(sh_c.at[pl.ds(sid * zlen, zlen)], bounce_v)

    @pl.when(cid == 0)
    def _():
        pltpu.sync_copy(bounce_v, cp0_hbm.at[pl.ds(sid * zlen, zlen)])

    @pl.when(cid == 1)
    def _():
        pltpu.sync_copy(bounce_v, cp1_hbm.at[pl.ds(sid * zlen, zlen)])


NBUF = 2
NIB = 4


@functools.partial(
    pl.kernel,
    out_type=(
        jax.ShapeDtypeStruct((N, D), jnp.float32),
        jax.ShapeDtypeStruct((N, D), jnp.float32),
    ),
    mesh=_mesh,
    scratch_types=[
        pltpu.VMEM((NIB, CKP), jnp.int32),
        pltpu.VMEM((NIB, CKP), jnp.int32),
        pltpu.VMEM((NBUF, CKP, D), jnp.float32),
        pltpu.VMEM((ZCK, D), jnp.float32),
        pltpu.VMEM_SHARED((NP, D), jnp.float32),
        [pltpu.SemaphoreType.DMA] * NIB,
        [pltpu.SemaphoreType.DMA] * NBUF,
        [pltpu.SemaphoreType.DMA] * NBUF,
    ],
)
def _sc_spmm(srcg_hbm, dstd_hbm, gp_hbm, zeros_nd_hbm,
             p0_hbm, p1_hbm,
             src_ring, dst_ring, rows, zb_v, sh_p, semi, semg, sems):
    cid = lax.axis_index("c")
    sid = lax.axis_index("s")
    wid = cid * NS + sid

    for j in range((NODE_CKS + NS - 1) // NS):
        ch = sid + j * NS

        @pl.when(ch < NODE_CKS)
        def _():
            pltpu.sync_copy(zeros_nd_hbm.at[pl.ds(ch * ZCK, ZCK)], zb_v)
            pltpu.sync_copy(zb_v, sh_p.at[pl.ds(ch * ZCK, ZCK)])

    plsc.subcore_barrier()

    def i_start(c, q):
        pltpu.async_copy(srcg_hbm.at[wid].at[c], src_ring.at[q], semi[q])
        pltpu.async_copy(dstd_hbm.at[wid].at[c], dst_ring.at[q], semi[q])

    def i_wait(c, q):
        pltpu.make_async_copy(srcg_hbm.at[wid].at[c], src_ring.at[q],
                              semi[q]).wait()
        pltpu.make_async_copy(dstd_hbm.at[wid].at[c], dst_ring.at[q],
                              semi[q]).wait()

    def g_start(c, q, b):
        pltpu.async_copy(gp_hbm.at[src_ring.at[q]], rows.at[b], semg[b])

    def g_wait(c, q, b):
        pltpu.make_async_copy(gp_hbm.at[src_ring.at[q]], rows.at[b],
                              semg[b]).wait()

    def s_start(c, q, b):
        pltpu.async_copy(rows.at[b], sh_p.at[dst_ring.at[q]], sems[b],
                         add=True)

    def s_wait(c, q, b):
        pltpu.make_async_copy(rows.at[b], sh_p.at[dst_ring.at[q]],
                              sems[b]).wait()

    i_start(0, 0)
    i_start(1, 1)
    i_wait(0, 0)
    g_start(0, 0, 0)

    def body(g, carry):
        for b4 in range(NIB):
            c = g * NIB + b4
            b = b4 % NBUF

            @pl.when(c + 2 < NCKP)
            def _():
                i_start(c + 2, (b4 + 2) % NIB)

            @pl.when(c - 1 >= 0)
            def _():
                s_wait(c - 1, (b4 + 3) % NIB, (b + 1) % NBUF)

            @pl.when(c + 1 < NCKP)
            def _():
                i_wait(c + 1, (b4 + 1) % NIB)
                g_start(c + 1, (b4 + 1) % NIB, (b + 1) % NBUF)

            g_wait(c, b4, b)
            s_start(c, b4, b)
        return carry

    lax.fori_loop(0, NCKP // NIB, body, 0)
    s_wait(NCKP - 1, (NCKP - 1) % NIB, (NCKP - 1) % NBUF)

    plsc.subcore_barrier()

    for j in range((NODE_CKS + NS - 1) // NS):
        ch = sid + j * NS

        @pl.when(ch < NODE_CKS)
        def _():
            pltpu.sync_copy(sh_p.at[pl.ds(ch * ZCK, ZCK)], zb_v)

            @pl.when(cid == 0)
            def _():
                pltpu.sync_copy(zb_v, p0_hbm.at[pl.ds(ch * ZCK, ZCK)])

            @pl.when(cid == 1)
            def _():
                pltpu.sync_copy(zb_v, p1_hbm.at[pl.ds(ch * ZCK, ZCK)])


def _tc_pre_body(d0_ref, d1_ref, h0_ref, w0_ref, dinv_ref, gp0_ref):
    dv = lax.rsqrt(d0_ref[...] + d1_ref[...] + 1.0)
    dinv_ref[...] = dv
    dnt = (((1,), (1,)), ((), ()))
    gp0_ref[...] = jnp.reshape(dv, (N, 1)) * lax.dot_general(
        h0_ref[...], w0_ref[...], dnt, preferred_element_type=jnp.float32)


def _tc_pre(deg0, deg1, h0, w0):
    return pl.pallas_call(
        _tc_pre_body,
        out_shape=[
            jax.ShapeDtypeStruct((N,), jnp.float32),
            jax.ShapeDtypeStruct((N, D), jnp.float32),
        ],
    )(deg0, deg1, h0, w0)


def _bn(o, g_ref, be_ref):
    mu = jnp.mean(o, axis=0, keepdims=True)
    var = jnp.mean((o - mu) ** 2, axis=0, keepdims=True)
    return g_ref[...] * (o - mu) * lax.rsqrt(var + BN_EPS) + be_ref[...]


def _edge_add(cp0_ref, cp1_ref, dv, ee_ref, b_ref):
    craw = cp0_ref[...] + cp1_ref[...]
    rp = jnp.sum(craw, axis=1, keepdims=True) + dv
    ee = ee_ref[...]
    dn = (((1,), (0,)), ((), ()))
    ce = lax.dot_general(craw, ee, dn,
                         preferred_element_type=jnp.float32) + dv * ee[0:1, :]
    return dv * ce + (dv * rp) * b_ref[...]


def _tc_mid_body(p0_ref, p1_ref, gp0_ref, cp0_ref, cp1_ref, dinv_ref,
                 ee0_ref, b0_ref, g0_ref, be0_ref, w1_ref, gp1_ref):
    dv = dinv_ref[...]
    o = dv * (p0_ref[...] + p1_ref[...] + gp0_ref[...]) + _edge_add(
        cp0_ref, cp1_ref, dv, ee0_ref, b0_ref)
    h = jax.nn.relu(_bn(o, g0_ref, be0_ref))
    dnt = (((1,), (1,)), ((), ()))
    gp1_ref[...] = dv * lax.dot_general(h, w1_ref[...], dnt,
                                        preferred_element_type=jnp.float32)


def _tc_mid(p0, p1, gp0, cp0, cp1, dinv2, ee0p, b0r, g0r, be0r, w1):
    return pl.pallas_call(
        _tc_mid_body,
        out_shape=jax.ShapeDtypeStruct((N, D), jnp.float32),
    )(p0, p1, gp0, cp0, cp1, dinv2, ee0p, b0r, g0r, be0r, w1)


def _tc_final_body(p0_ref, p1_ref, gp1_ref, cp0_ref, cp1_ref, dinv_ref,
                   ee1_ref, b1_ref, g1_ref, be1_ref, o_ref):
    dv = dinv_ref[...]
    o = dv * (p0_ref[...] + p1_ref[...] + gp1_ref[...]) + _edge_add(
        cp0_ref, cp1_ref, dv, ee1_ref, b1_ref)
    o_ref[...] = _bn(o, g1_ref, be1_ref)


def _tc_final(p0, p1, gp1, cp0, cp1, dinv2, ee1p, b1r, g1r, be1r):
    return pl.pallas_call(
        _tc_final_body,
        out_shape=jax.ShapeDtypeStruct((N, D), jnp.float32),
    )(p0, p1, gp1, cp0, cp1, dinv2, ee1p, b1r, g1r, be1r)


def kernel(x, edge_index, edge_attr, x_emb_table, W0, b0, ee0, g0, be0,
           W1, b1, ee1, g1, be1):
    src = edge_index[0]
    dst = edge_index[1]
    typ = edge_attr[:, 0]
    xcol = x[:, 0]
    npad = (E_PAD - E) // NW
    spread = jnp.arange(npad, dtype=src.dtype) % 128
    def _tiled(real, pad):
        padt = jnp.broadcast_to(pad, (NW, npad))
        return jnp.concatenate([real.reshape(NW, E // NW), padt],
                               axis=1).reshape(NW, NCKP, CKP)
    srcg = _tiled(src, spread)
    srcd = _tiled(src, N + (spread % 8))
    dstd = _tiled(dst, N + spread)
    cidx = _tiled(dst * T64 + typ, (N + spread) * T64)
    zeros_n = jnp.zeros((N,), jnp.float32)
    zeros_c = jnp.zeros((N * T64,), jnp.float32)
    zeros_nd = jnp.zeros((N, D), jnp.float32)
    ee0p = jnp.zeros((T64, D), jnp.float32).at[:ee0.shape[0]].set(ee0)
    ee1p = jnp.zeros((T64, D), jnp.float32).at[:ee1.shape[0]].set(ee1)

    deg0, deg1, h0 = _sc_deg_emb(srcd, xcol, x_emb_table, zeros_n)
    dinv, gp0 = _tc_pre(deg0, deg1, h0, W0)
    cp0f, cp1f = _sc_ctab(srcg, cidx, dinv, zeros_c)
    cp0 = cp0f.reshape(N, T64)
    cp1 = cp1f.reshape(N, T64)
    dinv2 = dinv[:, None]
    p0, p1 = _sc_spmm(srcg, dstd, gp0, zeros_nd)
    gp1 = _tc_mid(p0, p1, gp0, cp0, cp1, dinv2, ee0p, b0[None, :],
                  g0[None, :], be0[None, :], W1)
    q0, q1 = _sc_spmm(srcg, dstd, gp1, zeros_nd)
    return _tc_final(q0, q1, gp1, cp0, cp1, dinv2, ee1p, b1[None, :],
                     g1[None, :], be1[None, :])


# speedup vs baseline: 1.0419x; 1.0419x over previous
import functools

import jax
import jax.numpy as jnp
from jax import lax
from jax.experimental import pallas as pl
from jax.experimental.pallas import tpu as pltpu
from jax.experimental.pallas import tpu_sc as plsc

N = 10000
E = 320000
D = 128
T64 = 64
BN_EPS = 1e-5
NC, NS, L = 2, 16, 16
NW = NC * NS
CKP = 128
NCKP = 80
E_PAD = NW * NCKP * CKP
ZCK = 80
NODE_CKS = N // ZCK
NP = N + 128

_mesh = plsc.VectorSubcoreMesh(core_axis_name="c", subcore_axis_name="s")


@functools.partial(
    pl.kernel,
    out_type=(
        jax.ShapeDtypeStruct((N,), jnp.float32),
        jax.ShapeDtypeStruct((N,), jnp.float32),
        jax.ShapeDtypeStruct((N, D), jnp.float32),
    ),
    mesh=_mesh,
    scratch_types=[
        pltpu.VMEM((NCKP, CKP), jnp.int32),
        pltpu.VMEM((CKP,), jnp.float32),
        pltpu.VMEM((ZCK,), jnp.int32),
        pltpu.VMEM((ZCK, D), jnp.float32),
        pltpu.VMEM((N,), jnp.float32),
        pltpu.VMEM_SHARED((NP,), jnp.float32),
        pltpu.SemaphoreType.DMA,
        pltpu.SemaphoreType.DMA,
    ],
)
def _sc_deg_emb(srcd_hbm, xcol_hbm, emb_hbm, zeros_n_hbm,
                deg0_hbm, deg1_hbm, h0_hbm,
                src_all, ones_v, nid_v, rows_v, bounce_v, sh_deg, sems, semg):
    cid = lax.axis_index("c")
    sid = lax.axis_index("s")
    wid = cid * NS + sid

    pltpu.sync_copy(srcd_hbm.at[wid], src_all)

    @pl.when(sid == 0)
    def _():
        pltpu.sync_copy(zeros_n_hbm, bounce_v)
        pltpu.sync_copy(bounce_v, sh_deg.at[pl.ds(0, N)])

    for k in range(CKP // L):
        ones_v[pl.ds(k * L, L)] = jnp.ones((L,), jnp.float32)
    plsc.subcore_barrier()

    for grp in range(NCKP // 16):
        def fire(i, carry):
            c = grp * 16 + i
            pltpu.async_copy(ones_v, sh_deg.at[src_all.at[c]], sems, add=True)
            return carry

        lax.fori_loop(0, 16, fire, 0)

        def drain(i, carry):
            pltpu.make_async_copy(ones_v, sh_deg.at[src_all.at[0]], sems).wait()
            return carry

        lax.fori_loop(0, 16, drain, 0)

    for j in range((NODE_CKS + NW - 1) // NW):
        c = wid + j * NW

        @pl.when(c < NODE_CKS)
        def _():
            pltpu.sync_copy(xcol_hbm.at[pl.ds(c * ZCK, ZCK)], nid_v)
            pltpu.async_copy(emb_hbm.at[nid_v], rows_v, semg).wait()
            pltpu.sync_copy(rows_v, h0_hbm.at[pl.ds(c * ZCK, ZCK)])

    plsc.subcore_barrier()

    @pl.when((sid == 0) & (cid == 0))
    def _():
        pltpu.sync_copy(sh_deg.at[pl.ds(0, N)], bounce_v)
        pltpu.sync_copy(bounce_v, deg0_hbm)

    @pl.when((sid == 0) & (cid == 1))
    def _():
        pltpu.sync_copy(sh_deg.at[pl.ds(0, N)], bounce_v)
        pltpu.sync_copy(bounce_v, deg1_hbm)


@functools.partial(
    pl.kernel,
    out_type=(
        jax.ShapeDtypeStruct((N * T64,), jnp.float32),
        jax.ShapeDtypeStruct((N * T64,), jnp.float32),
    ),
    mesh=_mesh,
    scratch_types=[
        pltpu.VMEM((4, CKP), jnp.int32),
        pltpu.VMEM((4, CKP), jnp.int32),
        pltpu.VMEM((4, CKP), jnp.float32),
        pltpu.VMEM((N * T64 // NS,), jnp.float32),
        pltpu.VMEM_SHARED((NP * T64,), jnp.float32),
        [pltpu.SemaphoreType.DMA] * 4,
        [pltpu.SemaphoreType.DMA] * 4,
        [pltpu.SemaphoreType.DMA] * 4,
    ],
)
def _sc_ctab(srcg_hbm, cidx_hbm, dinv_hbm, zeros_c_hbm,
             cp0_hbm, cp1_hbm,
             src_ring, dst_ring, val_ring, bounce_v, sh_c,
             semi, semg, sems):
    cid = lax.axis_index("c")
    sid = lax.axis_index("s")
    wid = cid * NS + sid
    zlen = N * T64 // NS

    pltpu.sync_copy(zeros_c_hbm.at[pl.ds(sid * zlen, zlen)], bounce_v)
    pltpu.sync_copy(bounce_v, sh_c.at[pl.ds(sid * zlen, zlen)])
    plsc.subcore_barrier()

    def l_start(c, q):
        pltpu.async_copy(srcg_hbm.at[wid].at[c], src_ring.at[q], semi[q])
        pltpu.async_copy(cidx_hbm.at[wid].at[c], dst_ring.at[q], semi[q])

    def l_wait(c, q):
        for _ in range(2):
            pltpu.make_async_copy(srcg_hbm.at[wid].at[c], src_ring.at[q],
                                  semi[q]).wait()

    def g_start(c, q):
        pltpu.async_copy(dinv_hbm.at[src_ring.at[q]], val_ring.at[q], semg[q])

    def g_wait(c, q):
        pltpu.make_async_copy(dinv_hbm.at[src_ring.at[q]], val_ring.at[q],
                              semg[q]).wait()

    def s_start(c, q):
        pltpu.async_copy(val_ring.at[q], sh_c.at[dst_ring.at[q]], sems[q],
                         add=True)

    def s_wait(c, q):
        pltpu.make_async_copy(val_ring.at[q], sh_c.at[dst_ring.at[q]],
                              sems[q]).wait()

    l_start(0, 0)
    l_start(1, 1)
    l_wait(0, 0)
    g_start(0, 0)

    def body(g, carry):
        for q in range(4):
            c = g * 4 + q

            @pl.when(c - 2 >= 0)
            def _():
                s_wait(c - 2, (q + 2) % 4)

            @pl.when(c + 2 < NCKP)
            def _():
                l_start(c + 2, (q + 2) % 4)

            @pl.when(c + 1 < NCKP)
            def _():
                l_wait(c + 1, (q + 1) % 4)
                g_start(c + 1, (q + 1) % 4)

            g_wait(c, q)
            s_start(c, q)
        return carry

    lax.fori_loop(0, NCKP // 4, body, 0)
    s_wait(NCKP - 2, (NCKP - 2) % 4)
    s_wait(NCKP - 1, (NCKP - 1) % 4)

    plsc.subcore_barrier()
    pltpu.sync_copy(sh_c.at[pl.ds(sid * zlen, zlen)], bounce_v)

    @pl.when(cid == 0)
    def _():
        pltpu.sync_copy(bounce_v, cp0_hbm.at[pl.ds(sid * zlen, zlen)])

    @pl.when(cid == 1)
    def _():
        pltpu.sync_copy(bounce_v, cp1_hbm.at[pl.ds(sid * zlen, zlen)])


NBUF = 2
NIB = 4


@functools.partial(
    pl.kernel,
    out_type=(
        jax.ShapeDtypeStruct((N, D), jnp.float32),
        jax.ShapeDtypeStruct((N, D), jnp.float32),
    ),
    mesh=_mesh,
    scratch_types=[
        pltpu.VMEM((NIB, CKP), jnp.int32),
        pltpu.VMEM((NIB, CKP), jnp.int32),
        pltpu.VMEM((NBUF, CKP, D), jnp.float32),
        pltpu.VMEM((ZCK, D), jnp.float32),
        pltpu.VMEM_SHARED((NP, D), jnp.float32),
        [pltpu.SemaphoreType.DMA] * NIB,
        [pltpu.SemaphoreType.DMA] * NBUF,
        [pltpu.SemaphoreType.DMA] * NBUF,
    ],
)
def _sc_spmm(srcg_hbm, dstd_hbm, gp_hbm, zeros_nd_hbm,
             p0_hbm, p1_hbm,
             src_ring, dst_ring, rows, zb_v, sh_p, semi, semg, sems):
    cid = lax.axis_index("c")
    sid = lax.axis_index("s")
    wid = cid * NS + sid

    for j in range((NODE_CKS + NS - 1) // NS):
        ch = sid + j * NS

        @pl.when(ch < NODE_CKS)
        def _():
            pltpu.sync_copy(zeros_nd_hbm.at[pl.ds(ch * ZCK, ZCK)], zb_v)
            pltpu.sync_copy(zb_v, sh_p.at[pl.ds(ch * ZCK, ZCK)])

    plsc.subcore_barrier()

    def i_start(c, q):
        pltpu.async_copy(srcg_hbm.at[wid].at[c], src_ring.at[q], semi[q])
        pltpu.async_copy(dstd_hbm.at[wid].at[c], dst_ring.at[q], semi[q])

    def i_wait(c, q):
        pltpu.make_async_copy(srcg_hbm.at[wid].at[c], src_ring.at[q],
                              semi[q]).wait()
        pltpu.make_async_copy(dstd_hbm.at[wid].at[c], dst_ring.at[q],
                              semi[q]).wait()

    def g_start(c, q, b):
        pltpu.async_copy(gp_hbm.at[src_ring.at[q]], rows.at[b], semg[b])

    def g_wait(c, q, b):
        pltpu.make_async_copy(gp_hbm.at[src_ring.at[q]], rows.at[b],
                              semg[b]).wait()

    def s_start(c, q, b):
        pltpu.async_copy(rows.at[b], sh_p.at[dst_ring.at[q]], sems[b],
                         add=True)

    def s_wait(c, q, b):
        pltpu.make_async_copy(rows.at[b], sh_p.at[dst_ring.at[q]],
                              sems[b]).wait()

    i_start(0, 0)
    i_start(1, 1)
    i_wait(0, 0)
    g_start(0, 0, 0)

    def body(g, carry):
        for b4 in range(NIB):
            c = g * NIB + b4
            b = b4 % NBUF

            @pl.when(c + 2 < NCKP)
            def _():
                i_start(c + 2, (b4 + 2) % NIB)

            @pl.when(c - 1 >= 0)
            def _():
                s_wait(c - 1, (b4 + 3) % NIB, (b + 1) % NBUF)

            @pl.when(c + 1 < NCKP)
            def _():
                i_wait(c + 1, (b4 + 1) % NIB)
                g_start(c + 1, (b4 + 1) % NIB, (b + 1) % NBUF)

            g_wait(c, b4, b)
            s_start(c, b4, b)
        return carry

    lax.fori_loop(0, NCKP // NIB, body, 0)
    s_wait(NCKP - 1, (NCKP - 1) % NIB, (NCKP - 1) % NBUF)

    plsc.subcore_barrier()

    for j in range((NODE_CKS + NS - 1) // NS):
        ch = sid + j * NS

        @pl.when(ch < NODE_CKS)
        def _():
            pltpu.sync_copy(sh_p.at[pl.ds(ch * ZCK, ZCK)], zb_v)

            @pl.when(cid == 0)
            def _():
                pltpu.sync_copy(zb_v, p0_hbm.at[pl.ds(ch * ZCK, ZCK)])

            @pl.when(cid == 1)
            def _():
                pltpu.sync_copy(zb_v, p1_hbm.at[pl.ds(ch * ZCK, ZCK)])


def _tc_pre_body(d0_ref, d1_ref, h0_ref, w0_ref, dinv_ref, gp0_ref):
    dv = lax.rsqrt(d0_ref[...] + d1_ref[...] + 1.0)
    dinv_ref[...] = dv
    dnt = (((1,), (1,)), ((), ()))
    gp0_ref[...] = jnp.reshape(dv, (N, 1)) * lax.dot_general(
        h0_ref[...], w0_ref[...], dnt, preferred_element_type=jnp.float32)


def _tc_pre(deg0, deg1, h0, w0):
    return pl.pallas_call(
        _tc_pre_body,
        out_shape=[
            jax.ShapeDtypeStruct((N,), jnp.float32),
            jax.ShapeDtypeStruct((N, D), jnp.float32),
        ],
    )(deg0, deg1, h0, w0)


def _bn(o, g_ref, be_ref):
    mu = jnp.mean(o, axis=0, keepdims=True)
    var = jnp.mean((o - mu) ** 2, axis=0, keepdims=True)
    return g_ref[...] * (o - mu) * lax.rsqrt(var + BN_EPS) + be_ref[...]


def _edge_add(cp0_ref, cp1_ref, dv, ee_ref, b_ref):
    craw = cp0_ref[...] + cp1_ref[...]
    rp = jnp.sum(craw, axis=1, keepdims=True) + dv
    ee = ee_ref[...]
    dn = (((1,), (0,)), ((), ()))
    ce = lax.dot_general(craw, ee, dn,
                         preferred_element_type=jnp.float32) + dv * ee[0:1, :]
    return dv * ce + (dv * rp) * b_ref[...]


def _tc_mid_body(p0_ref, p1_ref, gp0_ref, cp0_ref, cp1_ref, dinv_ref,
                 ee0_ref, b0_ref, g0_ref, be0_ref, w1_ref, gp1_ref):
    dv = dinv_ref[...]
    o = dv * (p0_ref[...] + p1_ref[...] + gp0_ref[...]) + _edge_add(
        cp0_ref, cp1_ref, dv, ee0_ref, b0_ref)
    h = jax.nn.relu(_bn(o, g0_ref, be0_ref))
    dnt = (((1,), (1,)), ((), ()))
    gp1_ref[...] = dv * lax.dot_general(h, w1_ref[...], dnt,
                                        preferred_element_type=jnp.float32)


def _tc_mid(p0, p1, gp0, cp0, cp1, dinv2, ee0p, b0r, g0r, be0r, w1):
    return pl.pallas_call(
        _tc_mid_body,
        out_shape=jax.ShapeDtypeStruct((N, D), jnp.float32),
    )(p0, p1, gp0, cp0, cp1, dinv2, ee0p, b0r, g0r, be0r, w1)


def _tc_final_body(p0_ref, p1_ref, gp1_ref, cp0_ref, cp1_ref, dinv_ref,
                   ee1_ref, b1_ref, g1_ref, be1_ref, o_ref):
    dv = dinv_ref[...]
    o = dv * (p0_ref[...] + p1_ref[...] + gp1_ref[...]) + _edge_add(
        cp0_ref, cp1_ref, dv, ee1_ref, b1_ref)
    o_ref[...] = _bn(o, g1_ref, be1_ref)


def _tc_final(p0, p1, gp1, cp0, cp1, dinv2, ee1p, b1r, g1r, be1r):
    return pl.pallas_call(
        _tc_final_body,
        out_shape=jax.ShapeDtypeStruct((N, D), jnp.float32),
    )(p0, p1, gp1, cp0, cp1, dinv2, ee1p, b1r, g1r, be1r)


def kernel(x, edge_index, edge_attr, x_emb_table, W0, b0, ee0, g0, be0,
           W1, b1, ee1, g1, be1):
    src = edge_index[0]
    dst = edge_index[1]
    typ = edge_attr[:, 0]
    xcol = x[:, 0]
    npad = E_PAD - E
    spread = jnp.arange(npad, dtype=src.dtype) % 128
    srcg = jnp.concatenate([src, spread]).reshape(NW, NCKP, CKP)
    srcd = jnp.concatenate([src, N + (spread % 8)]).reshape(NW, NCKP, CKP)
    dstd = jnp.concatenate([dst, N + spread]).reshape(NW, NCKP, CKP)
    cidx = jnp.concatenate(
        [dst * T64 + typ, (N + spread) * T64]).reshape(NW, NCKP, CKP)
    zeros_n = jnp.zeros((N,), jnp.float32)
    zeros_c = jnp.zeros((N * T64,), jnp.float32)
    zeros_nd = jnp.zeros((N, D), jnp.float32)
    ee0p = jnp.zeros((T64, D), jnp.float32).at[:ee0.shape[0]].set(ee0)
    ee1p = jnp.zeros((T64, D), jnp.float32).at[:ee1.shape[0]].set(ee1)

    deg0, deg1, h0 = _sc_deg_emb(srcd, xcol, x_emb_table, zeros_n)
    dinv, gp0 = _tc_pre(deg0, deg1, h0, W0)
    cp0f, cp1f = _sc_ctab(srcg, cidx, dinv, zeros_c)
    cp0 = cp0f.reshape(N, T64)
    cp1 = cp1f.reshape(N, T64)
    dinv2 = dinv[:, None]
    p0, p1 = _sc_spmm(srcg, dstd, gp0, zeros_nd)
    gp1 = _tc_mid(p0, p1, gp0, cp0, cp1, dinv2, ee0p, b0[None, :],
                  g0[None, :], be0[None, :], W1)
    q0, q1 = _sc_spmm(srcg, dstd, gp1, zeros_nd)
    return _tc_final(q0, q1, gp1, cp0, cp1, dinv2, ee1p, b1[None, :],
                     g1[None, :], be1[None, :])


# speedup vs baseline: 1.1051x; 1.0607x over previous
import functools

import jax
import jax.numpy as jnp
from jax import lax
from jax.experimental import pallas as pl
from jax.experimental.pallas import tpu as pltpu
from jax.experimental.pallas import tpu_sc as plsc

N = 10000
E = 320000
D = 128
T64 = 64
BN_EPS = 1e-5
NC, NS, L = 2, 16, 16
NW = NC * NS
CKP = 128
NCKP = 80
E_PAD = NW * NCKP * CKP
ZCK = 80
NODE_CKS = N // ZCK
TAIL_CKS = (E - (NW - 1) * NCKP * CKP) // CKP
NP = N + 128

_mesh = plsc.VectorSubcoreMesh(core_axis_name="c", subcore_axis_name="s")


@functools.partial(
    pl.kernel,
    out_type=(
        jax.ShapeDtypeStruct((N,), jnp.float32),
        jax.ShapeDtypeStruct((N,), jnp.float32),
        jax.ShapeDtypeStruct((N, D), jnp.float32),
    ),
    mesh=_mesh,
    scratch_types=[
        pltpu.VMEM((NCKP, CKP), jnp.int32),
        pltpu.VMEM((CKP,), jnp.float32),
        pltpu.VMEM((ZCK,), jnp.int32),
        pltpu.VMEM((ZCK, D), jnp.float32),
        pltpu.VMEM((N,), jnp.float32),
        pltpu.VMEM_SHARED((NP,), jnp.float32),
        pltpu.SemaphoreType.DMA,
        pltpu.SemaphoreType.DMA,
    ],
)
def _sc_deg_emb(srcd_hbm, xcol_hbm, emb_hbm, zeros_n_hbm,
                deg0_hbm, deg1_hbm, h0_hbm,
                src_all, ones_v, nid_v, rows_v, bounce_v, sh_deg, sems, semg):
    cid = lax.axis_index("c")
    sid = lax.axis_index("s")
    wid = cid * NS + sid

    pltpu.sync_copy(srcd_hbm.at[wid], src_all)

    @pl.when(sid == 0)
    def _():
        pltpu.sync_copy(zeros_n_hbm, bounce_v)
        pltpu.sync_copy(bounce_v, sh_deg.at[pl.ds(0, N)])

    for k in range(CKP // L):
        ones_v[pl.ds(k * L, L)] = jnp.ones((L,), jnp.float32)
    plsc.subcore_barrier()

    nck = jnp.where(wid == NW - 1, TAIL_CKS, NCKP)
    for grp in range(NCKP // 16):
        def fire(i, carry):
            c = grp * 16 + i

            @pl.when(c < nck)
            def _():
                pltpu.async_copy(ones_v, sh_deg.at[src_all.at[c]], sems,
                                 add=True)
            return carry

        lax.fori_loop(0, 16, fire, 0)

        def drain(i, carry):
            @pl.when(grp * 16 + i < nck)
            def _():
                pltpu.make_async_copy(ones_v, sh_deg.at[src_all.at[0]],
                                      sems).wait()
            return carry

        lax.fori_loop(0, 16, drain, 0)

    for j in range((NODE_CKS + NW - 1) // NW):
        c = wid + j * NW

        @pl.when(c < NODE_CKS)
        def _():
            pltpu.sync_copy(xcol_hbm.at[pl.ds(c * ZCK, ZCK)], nid_v)
            pltpu.async_copy(emb_hbm.at[nid_v], rows_v, semg).wait()
            pltpu.sync_copy(rows_v, h0_hbm.at[pl.ds(c * ZCK, ZCK)])

    plsc.subcore_barrier()

    @pl.when((sid == 0) & (cid == 0))
    def _():
        pltpu.sync_copy(sh_deg.at[pl.ds(0, N)], bounce_v)
        pltpu.sync_copy(bounce_v, deg0_hbm)

    @pl.when((sid == 0) & (cid == 1))
    def _():
        pltpu.sync_copy(sh_deg.at[pl.ds(0, N)], bounce_v)
        pltpu.sync_copy(bounce_v, deg1_hbm)


@functools.partial(
    pl.kernel,
    out_type=(
        jax.ShapeDtypeStruct((N * T64,), jnp.float32),
        jax.ShapeDtypeStruct((N * T64,), jnp.float32),
    ),
    mesh=_mesh,
    scratch_types=[
        pltpu.VMEM((4, CKP), jnp.int32),
        pltpu.VMEM((4, CKP), jnp.int32),
        pltpu.VMEM((4, CKP), jnp.float32),
        pltpu.VMEM((N * T64 // NS,), jnp.float32),
        pltpu.VMEM_SHARED((NP * T64,), jnp.float32),
        [pltpu.SemaphoreType.DMA] * 4,
        [pltpu.SemaphoreType.DMA] * 4,
        [pltpu.SemaphoreType.DMA] * 4,
    ],
)
def _sc_ctab(srcg_hbm, cidx_hbm, dinv_hbm, zeros_c_hbm,
             cp0_hbm, cp1_hbm,
             src_ring, dst_ring, val_ring, bounce_v, sh_c,
             semi, semg, sems):
    cid = lax.axis_index("c")
    sid = lax.axis_index("s")
    wid = cid * NS + sid
    zlen = N * T64 // NS

    pltpu.sync_copy(zeros_c_hbm.at[pl.ds(sid * zlen, zlen)], bounce_v)
    pltpu.sync_copy(bounce_v, sh_c.at[pl.ds(sid * zlen, zlen)])
    plsc.subcore_barrier()

    def l_start(c, q):
        pltpu.async_copy(srcg_hbm.at[wid].at[c], src_ring.at[q], semi[q])
        pltpu.async_copy(cidx_hbm.at[wid].at[c], dst_ring.at[q], semi[q])

    def l_wait(c, q):
        for _ in range(2):
            pltpu.make_async_copy(srcg_hbm.at[wid].at[c], src_ring.at[q],
                                  semi[q]).wait()

    def g_start(c, q):
        pltpu.async_copy(dinv_hbm.at[src_ring.at[q]], val_ring.at[q], semg[q])

    def g_wait(c, q):
        pltpu.make_async_copy(dinv_hbm.at[src_ring.at[q]], val_ring.at[q],
                              semg[q]).wait()

    def s_start(c, q):
        pltpu.async_copy(val_ring.at[q], sh_c.at[dst_ring.at[q]], sems[q],
                         add=True)

    def s_wait(c, q):
        pltpu.make_async_copy(val_ring.at[q], sh_c.at[dst_ring.at[q]],
                              sems[q]).wait()

    l_start(0, 0)
    l_start(1, 1)
    l_wait(0, 0)
    g_start(0, 0)

    nck = jnp.where(wid == NW - 1, TAIL_CKS, NCKP)

    def body(g, carry):
        for q in range(4):
            c = g * 4 + q

            @pl.when(c < nck)
            def _():
                @pl.when(c - 2 >= 0)
                def _():
                    s_wait(c - 2, (q + 2) % 4)

                @pl.when(c + 2 < nck)
                def _():
                    l_start(c + 2, (q + 2) % 4)

                @pl.when(c + 1 < nck)
                def _():
                    l_wait(c + 1, (q + 1) % 4)
                    g_start(c + 1, (q + 1) % 4)

                g_wait(c, q)
                s_start(c, q)
        return carry

    lax.fori_loop(0, NCKP // 4, body, 0)
    s_wait(NCKP - 2, (NCKP - 2) % 4)
    s_wait(NCKP - 1, (NCKP - 1) % 4)

    plsc.subcore_barrier()
    pltpu.sync_copy(sh_c.at[pl.ds(sid * zlen, zlen)], bounce_v)

    @pl.when(cid == 0)
    def _():
        pltpu.sync_copy(bounce_v, cp0_hbm.at[pl.ds(sid * zlen, zlen)])

    @pl.when(cid == 1)
    def _():
        pltpu.sync_copy(bounce_v, cp1_hbm.at[pl.ds(sid * zlen, zlen)])


NBUF = 2
NIB = 4


@functools.partial(
    pl.kernel,
    out_type=(
        jax.ShapeDtypeStruct((N, D), jnp.float32),
        jax.ShapeDtypeStruct((N, D), jnp.float32),
    ),
    mesh=_mesh,
    scratch_types=[
        pltpu.VMEM((NIB, CKP), jnp.int32),
        pltpu.VMEM((NIB, CKP), jnp.int32),
        pltpu.VMEM((NBUF, CKP, D), jnp.float32),
        pltpu.VMEM((ZCK, D), jnp.float32),
        pltpu.VMEM_SHARED((NP, D), jnp.float32),
        [pltpu.SemaphoreType.DMA] * NIB,
        [pltpu.SemaphoreType.DMA] * NBUF,
        [pltpu.SemaphoreType.DMA] * NBUF,
    ],
)
def _sc_spmm(srcg_hbm, dstd_hbm, gp_hbm, zeros_nd_hbm,
             p0_hbm, p1_hbm,
             src_ring, dst_ring, rows, zb_v, sh_p, semi, semg, sems):
    cid = lax.axis_index("c")
    sid = lax.axis_index("s")
    wid = cid * NS + sid

    for j in range((NODE_CKS + NS - 1) // NS):
        ch = sid + j * NS

        @pl.when(ch < NODE_CKS)
        def _():
            pltpu.sync_copy(zeros_nd_hbm.at[pl.ds(ch * ZCK, ZCK)], zb_v)
            pltpu.sync_copy(zb_v, sh_p.at[pl.ds(ch * ZCK, ZCK)])

    plsc.subcore_barrier()

    def i_start(c, q):
        pltpu.async_copy(srcg_hbm.at[wid].at[c], src_ring.at[q], semi[q])
        pltpu.async_copy(dstd_hbm.at[wid].at[c], dst_ring.at[q], semi[q])

    def i_wait(c, q):
        pltpu.make_async_copy(srcg_hbm.at[wid].at[c], src_ring.at[q],
                              semi[q]).wait()
        pltpu.make_async_copy(dstd_hbm.at[wid].at[c], dst_ring.at[q],
                              semi[q]).wait()

    def g_start(c, q, b):
        pltpu.async_copy(gp_hbm.at[src_ring.at[q]], rows.at[b], semg[b])

    def g_wait(c, q, b):
        pltpu.make_async_copy(gp_hbm.at[src_ring.at[q]], rows.at[b],
                              semg[b]).wait()

    def s_start(c, q, b):
        pltpu.async_copy(rows.at[b], sh_p.at[dst_ring.at[q]], sems[b],
                         add=True)

    def s_wait(c, q, b):
        pltpu.make_async_copy(rows.at[b], sh_p.at[dst_ring.at[q]],
                              sems[b]).wait()

    i_start(0, 0)
    i_start(1, 1)
    i_wait(0, 0)
    g_start(0, 0, 0)

    nck = jnp.where(wid == NW - 1, TAIL_CKS, NCKP)

    def body(g, carry):
        for b4 in range(NIB):
            c = g * NIB + b4
            b = b4 % NBUF

            @pl.when(c < nck)
            def _():
                @pl.when(c + 2 < nck)
                def _():
                    i_start(c + 2, (b4 + 2) % NIB)

                @pl.when(c - 1 >= 0)
                def _():
                    s_wait(c - 1, (b4 + 3) % NIB, (b + 1) % NBUF)

                @pl.when(c + 1 < nck)
                def _():
                    i_wait(c + 1, (b4 + 1) % NIB)
                    g_start(c + 1, (b4 + 1) % NIB, (b + 1) % NBUF)

                g_wait(c, b4, b)
                s_start(c, b4, b)
        return carry

    lax.fori_loop(0, NCKP // NIB, body, 0)
    s_wait(NCKP - 1, (NCKP - 1) % NIB, (NCKP - 1) % NBUF)

    plsc.subcore_barrier()

    for j in range((NODE_CKS + NS - 1) // NS):
        ch = sid + j * NS

        @pl.when(ch < NODE_CKS)
        def _():
            pltpu.sync_copy(sh_p.at[pl.ds(ch * ZCK, ZCK)], zb_v)

            @pl.when(cid == 0)
            def _():
                pltpu.sync_copy(zb_v, p0_hbm.at[pl.ds(ch * ZCK, ZCK)])

            @pl.when(cid == 1)
            def _():
                pltpu.sync_copy(zb_v, p1_hbm.at[pl.ds(ch * ZCK, ZCK)])


def _tc_pre_body(d0_ref, d1_ref, h0_ref, w0_ref, dinv_ref, gp0_ref):
    dv = lax.rsqrt(d0_ref[...] + d1_ref[...] + 1.0)
    dinv_ref[...] = dv
    dnt = (((1,), (1,)), ((), ()))
    gp0_ref[...] = jnp.reshape(dv, (N, 1)) * lax.dot_general(
        h0_ref[...], w0_ref[...], dnt, preferred_element_type=jnp.float32)


def _tc_pre(deg0, deg1, h0, w0):
    return pl.pallas_call(
        _tc_pre_body,
        out_shape=[
            jax.ShapeDtypeStruct((N,), jnp.float32),
            jax.ShapeDtypeStruct((N, D), jnp.float32),
        ],
    )(deg0, deg1, h0, w0)


def _bn(o, g_ref, be_ref):
    mu = jnp.mean(o, axis=0, keepdims=True)
    var = jnp.mean((o - mu) ** 2, axis=0, keepdims=True)
    return g_ref[...] * (o - mu) * lax.rsqrt(var + BN_EPS) + be_ref[...]


def _edge_add(cp0_ref, cp1_ref, dv, ee_ref, b_ref):
    craw = cp0_ref[...] + cp1_ref[...]
    rp = jnp.sum(craw, axis=1, keepdims=True) + dv
    ee = ee_ref[...]
    dn = (((1,), (0,)), ((), ()))
    ce = lax.dot_general(craw, ee, dn,
                         preferred_element_type=jnp.float32) + dv * ee[0:1, :]
    return dv * ce + (dv * rp) * b_ref[...]


def _tc_mid_body(p0_ref, p1_ref, gp0_ref, cp0_ref, cp1_ref, dinv_ref,
                 ee0_ref, b0_ref, g0_ref, be0_ref, w1_ref, gp1_ref):
    dv = dinv_ref[...]
    o = dv * (p0_ref[...] + p1_ref[...] + gp0_ref[...]) + _edge_add(
        cp0_ref, cp1_ref, dv, ee0_ref, b0_ref)
    h = jax.nn.relu(_bn(o, g0_ref, be0_ref))
    dnt = (((1,), (1,)), ((), ()))
    gp1_ref[...] = dv * lax.dot_general(h, w1_ref[...], dnt,
                                        preferred_element_type=jnp.float32)


def _tc_mid(p0, p1, gp0, cp0, cp1, dinv2, ee0p, b0r, g0r, be0r, w1):
    return pl.pallas_call(
        _tc_mid_body,
        out_shape=jax.ShapeDtypeStruct((N, D), jnp.float32),
    )(p0, p1, gp0, cp0, cp1, dinv2, ee0p, b0r, g0r, be0r, w1)


def _tc_final_body(p0_ref, p1_ref, gp1_ref, cp0_ref, cp1_ref, dinv_ref,
                   ee1_ref, b1_ref, g1_ref, be1_ref, o_ref):
    dv = dinv_ref[...]
    o = dv * (p0_ref[...] + p1_ref[...] + gp1_ref[...]) + _edge_add(
        cp0_ref, cp1_ref, dv, ee1_ref, b1_ref)
    o_ref[...] = _bn(o, g1_ref, be1_ref)


def _tc_final(p0, p1, gp1, cp0, cp1, dinv2, ee1p, b1r, g1r, be1r):
    return pl.pallas_call(
        _tc_final_body,
        out_shape=jax.ShapeDtypeStruct((N, D), jnp.float32),
    )(p0, p1, gp1, cp0, cp1, dinv2, ee1p, b1r, g1r, be1r)


def kernel(x, edge_index, edge_attr, x_emb_table, W0, b0, ee0, g0, be0,
           W1, b1, ee1, g1, be1):
    src = edge_index[0]
    dst = edge_index[1]
    typ = edge_attr[:, 0]
    xcol = x[:, 0]
    npad = E_PAD - E
    spread = jnp.arange(npad, dtype=src.dtype) % 128
    srcg = jnp.concatenate([src, spread]).reshape(NW, NCKP, CKP)
    srcd = jnp.concatenate([src, N + (spread % 8)]).reshape(NW, NCKP, CKP)
    dstd = jnp.concatenate([dst, N + spread]).reshape(NW, NCKP, CKP)
    cidx = jnp.concatenate(
        [dst * T64 + typ, (N + spread) * T64]).reshape(NW, NCKP, CKP)
    zeros_n = jnp.zeros((N,), jnp.float32)
    zeros_c = jnp.zeros((N * T64,), jnp.float32)
    zeros_nd = jnp.zeros((N, D), jnp.float32)
    ee0p = jnp.zeros((T64, D), jnp.float32).at[:ee0.shape[0]].set(ee0)
    ee1p = jnp.zeros((T64, D), jnp.float32).at[:ee1.shape[0]].set(ee1)

    deg0, deg1, h0 = _sc_deg_emb(srcd, xcol, x_emb_table, zeros_n)
    dinv, gp0 = _tc_pre(deg0, deg1, h0, W0)
    cp0f, cp1f = _sc_ctab(srcg, cidx, dinv, zeros_c)
    cp0 = cp0f.reshape(N, T64)
    cp1 = cp1f.reshape(N, T64)
    dinv2 = dinv[:, None]
    p0, p1 = _sc_spmm(srcg, dstd, gp0, zeros_nd)
    gp1 = _tc_mid(p0, p1, gp0, cp0, cp1, dinv2, ee0p, b0[None, :],
                  g0[None, :], be0[None, :], W1)
    q0, q1 = _sc_spmm(srcg, dstd, gp1, zeros_nd)
    return _tc_final(q0, q1, gp1, cp0, cp1, dinv2, ee1p, b1[None, :],
                     g1[None, :], be1[None, :])
